# baseline (speedup 1.0000x reference)
import functools

import jax
import jax.numpy as jnp
from jax import lax
from jax.experimental import pallas as pl
from jax.experimental.pallas import tpu as pltpu
from jax.experimental.pallas import tpu_sc as plsc

EPS = 1e-15
N_RBF = 20
CUTOFF = 5.0
F = 128

N_NODES = 10000
N_EDGES = 320000

NW = 32
PASSES = 3
NSEG = NW * PASSES
NLOC = 105
CAP = 4224
CH = 3200
GB = 16


def _splat_gather(x, idx16):
    return x.at[idx16].get(mode="promise_in_bounds")


def _node_body(s_ref, vjt_ref, w1_ref, b1_ref, w2_ref, b2_ref, o_ref):
    s = s_ref[...]
    h = jnp.dot(s, w1_ref[...], preferred_element_type=jnp.float32) + b1_ref[...]
    h = h * jax.nn.sigmoid(h)
    phi = jnp.dot(h, w2_ref[...], preferred_element_type=jnp.float32) + b2_ref[...]
    o_ref[...] = jnp.concatenate([phi, vjt_ref[...]], axis=1)


def _node_table(s_j, vjt, W1, b1, W2, b2):
    n = s_j.shape[0]
    B = 2000
    return pl.pallas_call(
        _node_body,
        grid=(n // B,),
        in_specs=[
            pl.BlockSpec((B, F), lambda i: (i, 0)),
            pl.BlockSpec((B, 3 * F), lambda i: (i, 0)),
            pl.BlockSpec((F, F), lambda i: (0, 0)),
            pl.BlockSpec((1, F), lambda i: (0, 0)),
            pl.BlockSpec((F, 3 * F), lambda i: (0, 0)),
            pl.BlockSpec((1, 3 * F), lambda i: (0, 0)),
        ],
        out_specs=pl.BlockSpec((B, 6 * F), lambda i: (i, 0)),
        out_shape=jax.ShapeDtypeStruct((n, 6 * F), jnp.float32),
    )(s_j, vjt, W1, b1[None, :], W2, b2[None, :])


def _edge_body(r_ref, wd_ref, bd_ref, o_ref):
    r = r_ref[...]
    dist = jnp.sqrt(((r * r) + EPS).sum(-1))
    unit = r / dist[:, None]
    n = lax.iota(jnp.int32, N_RBF).astype(jnp.float32) + 1.0
    coef = n * (jnp.pi / CUTOFF)
    rbf = jnp.sin(coef * dist[:, None]) / dist[:, None]
    rbf_feats = jnp.dot(rbf, wd_ref[...], preferred_element_type=jnp.float32) + bd_ref[...]
    env = jnp.where(dist <= CUTOFF, 0.5 * (jnp.cos(jnp.pi / CUTOFF * dist) + 1.0), 0.0)
    w_s = rbf_feats * env[:, None]
    ws2 = w_s[:, 2 * F:3 * F]
    o_ref[...] = jnp.concatenate(
        [w_s[:, 0:F], w_s[:, F:2 * F],
         ws2 * unit[:, 0:1], ws2 * unit[:, 1:2], ws2 * unit[:, 2:3]], axis=1)


def _edge_table(r_ij, Wd, bd):
    e = r_ij.shape[0]
    B = 2000
    return pl.pallas_call(
        _edge_body,
        grid=(e // B,),
        in_specs=[
            pl.BlockSpec((B, 3), lambda i: (i, 0)),
            pl.BlockSpec((N_RBF, 3 * F), lambda i: (0, 0)),
            pl.BlockSpec((1, 3 * F), lambda i: (0, 0)),
        ],
        out_specs=pl.BlockSpec((B, 5 * F), lambda i: (i, 0)),
        out_shape=jax.ShapeDtypeStruct((e, 5 * F), jnp.float32),
    )(r_ij, Wd, bd[None, :])


def _sc_body(nt_hbm, we_hbm, dst_hbm, src_hbm, ds_hbm, dv_hbm,
             acc_s, acc_v, ids_l, src_l, dst_l, st_dst, st_src,
             nt_buf, we_buf, sem):
    nc = 2
    wid = lax.axis_index("s") * nc + lax.axis_index("c")
    zs = jnp.zeros((16,), jnp.float32)
    zi = jnp.zeros((16,), jnp.int32)
    iota16 = lax.iota(jnp.int32, 16)
    last16 = jnp.full((16,), 15, jnp.int32)

    def one_pass(p, _):
        seg = wid * PASSES + p
        segbase = seg * NLOC

        def zs_body(i, _):
            acc_s[pl.ds(i * 16, 16)] = zs
            return 0
        lax.fori_loop(0, (NLOC + 1) * F // 16, zs_body, 0)

        def zv_body(i, _):
            acc_v[pl.ds(i * 16, 16)] = zs
            return 0
        lax.fori_loop(0, (NLOC + 1) * 3 * F // 16, zv_body, 0)

        trash = jnp.full((16,), NLOC, jnp.int32)

        def pf_body(i, _):
            ids_l[pl.ds(i * 16, 16)] = zi
            src_l[pl.ds(i * 16, 16)] = zi
            dst_l[pl.ds(i * 16, 16)] = trash
            return 0
        lax.fori_loop(0, CAP // 16, pf_body, 0)

        lo = segbase
        hi = segbase + NLOC

        def chunk_body(c, cnt):
            pltpu.sync_copy(dst_hbm.at[pl.ds(c * CH, CH)], st_dst)
            pltpu.sync_copy(src_hbm.at[pl.ds(c * CH, CH)], st_src)

            def grp_body(g, cnt):
                for u in range(4):
                    goff = (g * 4 + u) * 16
                    dstv = st_dst[pl.ds(goff, 16)]
                    m = (dstv >= lo) & (dstv < hi)
                    w1 = jnp.where(m, 1, 0).astype(jnp.int32)
                    tot = w1
                    for sh in (1, 2, 4, 8):
                        tot = tot + _splat_gather(tot, iota16 ^ sh)

                    @pl.when(tot[0] > 0)
                    def _(goff=goff, m=m, w1=w1, cnt=cnt):
                        srcv = st_src[pl.ds(goff, 16)]
                        eidv = iota16 + (c * CH + goff)
                        locv = jnp.where(m, dstv - lo, NLOC).astype(jnp.int32)
                        off = cnt[0]
                        for j in range(16):
                            fj = jnp.full((16,), j, jnp.int32)

                            @pl.when(w1[j] == 1)
                            def _(off=off, fj=fj):
                                ids_l[pl.ds(off, 16)] = _splat_gather(eidv, fj)
                                src_l[pl.ds(off, 16)] = _splat_gather(srcv, fj)
                                dst_l[pl.ds(off, 16)] = _splat_gather(locv, fj)
                            off = off + w1[j]
                        ids_l[pl.ds(off, 16)] = zi
                        src_l[pl.ds(off, 16)] = zi
                        dst_l[pl.ds(off, 16)] = trash
                    cnt = jnp.minimum(cnt + tot, CAP - 16)
                return cnt

            return lax.fori_loop(0, CH // 64, grp_body, cnt)

        cnt = lax.fori_loop(0, N_EDGES // CH, chunk_body, zi)
        cs = cnt[0]
        nb = (cs + (GB - 1)) // GB

        def batch_body(j, _):
            j0 = j * GB
            srcs = src_l[pl.ds(j0, GB)]
            eids = ids_l[pl.ds(j0, GB)]
            locv = dst_l[pl.ds(j0, GB)]
            cp1 = pltpu.async_copy(nt_hbm.at[srcs], nt_buf, sem)
            cp2 = pltpu.async_copy(we_hbm.at[eids], we_buf, sem)
            cp1.wait()
            cp2.wait()
            for e in range(GB):
                le = locv[e]
                for k in range(8):
                    k16 = k * 16
                    phi0 = nt_buf[e, pl.ds(k16, 16)]
                    phi1 = nt_buf[e, pl.ds(F + k16, 16)]
                    phi2 = nt_buf[e, pl.ds(2 * F + k16, 16)]
                    ws0 = we_buf[e, pl.ds(k16, 16)]
                    ws1 = we_buf[e, pl.ds(F + k16, 16)]
                    plsc.addupdate(acc_s.at[pl.ds(le * F + k16, 16)],
                                   phi1 * ws1)
                    t0 = phi0 * ws0
                    for a in range(3):
                        w2u = we_buf[e, pl.ds(2 * F + a * F + k16, 16)]
                        vja = nt_buf[e, pl.ds(3 * F + a * F + k16, 16)]
                        plsc.addupdate(
                            acc_v.at[pl.ds(le * (3 * F) + a * F + k16, 16)],
                            phi2 * w2u + t0 * vja)
            return 0

        lax.fori_loop(0, nb, batch_body, 0)

        pltpu.sync_copy(acc_s.at[pl.ds(0, NLOC * F)],
                        ds_hbm.at[pl.ds(segbase * F, NLOC * F)])
        pltpu.sync_copy(acc_v.at[pl.ds(0, NLOC * 3 * F)],
                        dv_hbm.at[pl.ds(segbase * 3 * F, NLOC * 3 * F)])
        return 0

    lax.fori_loop(0, PASSES, one_pass, 0)


def _sc_call(nt, we, dst, src):
    mesh = plsc.VectorSubcoreMesh(core_axis_name="c", subcore_axis_name="s")
    kfn = functools.partial(
        pl.kernel, mesh=mesh,
        out_type=[
            jax.ShapeDtypeStruct((NSEG * NLOC * F,), jnp.float32),
            jax.ShapeDtypeStruct((NSEG * NLOC * 3 * F,), jnp.float32),
        ],
        scratch_types=[
            pltpu.VMEM(((NLOC + 1) * F,), jnp.float32),
            pltpu.VMEM(((NLOC + 1) * 3 * F,), jnp.float32),
            pltpu.VMEM((CAP + 16,), jnp.int32),
            pltpu.VMEM((CAP + 16,), jnp.int32),
            pltpu.VMEM((CAP + 16,), jnp.int32),
            pltpu.VMEM((CH,), jnp.int32),
            pltpu.VMEM((CH,), jnp.int32),
            pltpu.VMEM((GB, 6 * F), jnp.float32),
            pltpu.VMEM((GB, 5 * F), jnp.float32),
            pltpu.SemaphoreType.DMA,
        ],
    )(_sc_body)
    return kfn(nt, we, dst, src)


def kernel(s_j, v_j, r_ij, nbrs, W1, b1, W2, b2, Wd, bd):
    vjt = v_j.swapaxes(1, 2).reshape(N_NODES, 3 * F)
    nt = _node_table(s_j, vjt, W1, b1, W2, b2)
    we = _edge_table(r_ij, Wd, bd)
    dst = nbrs[:, 0]
    src = nbrs[:, 1]
    ds_flat, dv_flat = _sc_call(nt, we, dst, src)
    delta_s = ds_flat.reshape(NSEG * NLOC, F)[:N_NODES]
    delta_v = dv_flat.reshape(NSEG * NLOC, 3, F)[:N_NODES].swapaxes(1, 2)
    return (delta_s, delta_v)


# speedup vs baseline: 1.1329x; 1.1329x over previous
import functools

import jax
import jax.numpy as jnp
from jax import lax
from jax.experimental import pallas as pl
from jax.experimental.pallas import tpu as pltpu
from jax.experimental.pallas import tpu_sc as plsc

EPS = 1e-15
N_RBF = 20
CUTOFF = 5.0
F = 128

N_NODES = 10000
N_EDGES = 320000

NW = 32
PASSES = 3
NSEG = NW * PASSES
NLOC = 105
CAP = 4224
CH = 6400
GB = 16


def _splat_gather(x, idx16):
    return x.at[idx16].get(mode="promise_in_bounds")


def _node_body(s_ref, vjt_ref, w1_ref, b1_ref, w2_ref, b2_ref, o_ref):
    s = s_ref[...]
    h = jnp.dot(s, w1_ref[...], preferred_element_type=jnp.float32) + b1_ref[...]
    h = h * jax.nn.sigmoid(h)
    phi = jnp.dot(h, w2_ref[...], preferred_element_type=jnp.float32) + b2_ref[...]
    o_ref[...] = jnp.concatenate([phi, vjt_ref[...]], axis=1)


def _node_table(s_j, vjt, W1, b1, W2, b2):
    n = s_j.shape[0]
    B = 2000
    return pl.pallas_call(
        _node_body,
        grid=(n // B,),
        in_specs=[
            pl.BlockSpec((B, F), lambda i: (i, 0)),
            pl.BlockSpec((B, 3 * F), lambda i: (i, 0)),
            pl.BlockSpec((F, F), lambda i: (0, 0)),
            pl.BlockSpec((1, F), lambda i: (0, 0)),
            pl.BlockSpec((F, 3 * F), lambda i: (0, 0)),
            pl.BlockSpec((1, 3 * F), lambda i: (0, 0)),
        ],
        out_specs=pl.BlockSpec((B, 6 * F), lambda i: (i, 0)),
        out_shape=jax.ShapeDtypeStruct((n, 6 * F), jnp.float32),
    )(s_j, vjt, W1, b1[None, :], W2, b2[None, :])


def _edge_body(r_ref, wd_ref, bd_ref, o_ref):
    r = r_ref[...]
    dist = jnp.sqrt(((r * r) + EPS).sum(-1))
    unit = r / dist[:, None]
    n = lax.iota(jnp.int32, N_RBF).astype(jnp.float32) + 1.0
    coef = n * (jnp.pi / CUTOFF)
    rbf = jnp.sin(coef * dist[:, None]) / dist[:, None]
    rbf_feats = jnp.dot(rbf, wd_ref[...], preferred_element_type=jnp.float32) + bd_ref[...]
    env = jnp.where(dist <= CUTOFF, 0.5 * (jnp.cos(jnp.pi / CUTOFF * dist) + 1.0), 0.0)
    w_s = rbf_feats * env[:, None]
    ws2 = w_s[:, 2 * F:3 * F]
    o_ref[...] = jnp.concatenate(
        [w_s[:, 0:F], w_s[:, F:2 * F],
         ws2 * unit[:, 0:1], ws2 * unit[:, 1:2], ws2 * unit[:, 2:3]], axis=1)


def _edge_table(r_ij, Wd, bd):
    e = r_ij.shape[0]
    B = 2000
    return pl.pallas_call(
        _edge_body,
        grid=(e // B,),
        in_specs=[
            pl.BlockSpec((B, 3), lambda i: (i, 0)),
            pl.BlockSpec((N_RBF, 3 * F), lambda i: (0, 0)),
            pl.BlockSpec((1, 3 * F), lambda i: (0, 0)),
        ],
        out_specs=pl.BlockSpec((B, 5 * F), lambda i: (i, 0)),
        out_shape=jax.ShapeDtypeStruct((e, 5 * F), jnp.float32),
    )(r_ij, Wd, bd[None, :])


def _sc_body(nt_hbm, we_hbm, dst_hbm, src_hbm, ds_hbm, dv_hbm,
             acc_s, acc_v, ids_l, src_l, dst_l, st_dst, st_src,
             nt_a, we_a, nt_b, we_b, sem_a, sem_b):
    nc = 2
    wid = lax.axis_index("s") * nc + lax.axis_index("c")
    zs = jnp.zeros((16,), jnp.float32)
    zi = jnp.zeros((16,), jnp.int32)
    iota16 = lax.iota(jnp.int32, 16)
    last16 = jnp.full((16,), 15, jnp.int32)

    def one_pass(p, _):
        seg = wid * PASSES + p
        segbase = seg * NLOC

        def zs_body(i, _):
            acc_s[pl.ds(i * 16, 16)] = zs
            return 0
        lax.fori_loop(0, (NLOC + 1) * F // 16, zs_body, 0)

        def zv_body(i, _):
            acc_v[pl.ds(i * 16, 16)] = zs
            return 0
        lax.fori_loop(0, (NLOC + 1) * 3 * F // 16, zv_body, 0)

        trash = jnp.full((16,), NLOC, jnp.int32)

        def pf_body(i, _):
            ids_l[pl.ds(i * 16, 16)] = zi
            src_l[pl.ds(i * 16, 16)] = zi
            dst_l[pl.ds(i * 16, 16)] = trash
            return 0
        lax.fori_loop(0, (CAP + 48) // 16, pf_body, 0)

        lo = segbase
        hi = segbase + NLOC

        def chunk_body(c, cnt):
            pltpu.sync_copy(dst_hbm.at[pl.ds(c * CH, CH)], st_dst)
            pltpu.sync_copy(src_hbm.at[pl.ds(c * CH, CH)], st_src)

            def grp_body(g, cnt):
                for u in range(4):
                    goff = (g * 4 + u) * 16
                    dstv = st_dst[pl.ds(goff, 16)]
                    m = (dstv >= lo) & (dstv < hi)
                    w1 = jnp.where(m, 1, 0).astype(jnp.int32)
                    tot = w1
                    for sh in (1, 2, 4, 8):
                        tot = tot + _splat_gather(tot, iota16 ^ sh)

                    @pl.when(tot[0] > 0)
                    def _(goff=goff, m=m, w1=w1, cnt=cnt):
                        srcv = st_src[pl.ds(goff, 16)]
                        eidv = iota16 + (c * CH + goff)
                        locv = jnp.where(m, dstv - lo, NLOC).astype(jnp.int32)
                        off = cnt[0]
                        for j in range(16):
                            fj = jnp.full((16,), j, jnp.int32)

                            @pl.when(w1[j] == 1)
                            def _(off=off, fj=fj):
                                ids_l[pl.ds(off, 16)] = _splat_gather(eidv, fj)
                                src_l[pl.ds(off, 16)] = _splat_gather(srcv, fj)
                                dst_l[pl.ds(off, 16)] = _splat_gather(locv, fj)
                            off = off + w1[j]
                        ids_l[pl.ds(off, 16)] = zi
                        src_l[pl.ds(off, 16)] = zi
                        dst_l[pl.ds(off, 16)] = trash
                    cnt = jnp.minimum(cnt + tot, CAP - 16)
                return cnt

            return lax.fori_loop(0, CH // 64, grp_body, cnt)

        cnt = lax.fori_loop(0, N_EDGES // CH, chunk_body, zi)
        cs = cnt[0]
        nb = (cs + (GB - 1)) // GB

        def issue(j, ntb, web, sem):
            j0 = j * GB
            srcs = src_l[pl.ds(j0, GB)]
            eids = ids_l[pl.ds(j0, GB)]
            c1 = pltpu.async_copy(nt_hbm.at[srcs], ntb, sem)
            c2 = pltpu.async_copy(we_hbm.at[eids], web, sem)
            return c1, c2

        def compute(j, ntb, web):
            locv = dst_l[pl.ds(j * GB, GB)]
            for e in range(GB):
                le = locv[e]
                for k in range(8):
                    k16 = k * 16
                    phi0 = ntb[e, pl.ds(k16, 16)]
                    phi1 = ntb[e, pl.ds(F + k16, 16)]
                    phi2 = ntb[e, pl.ds(2 * F + k16, 16)]
                    ws0 = web[e, pl.ds(k16, 16)]
                    ws1 = web[e, pl.ds(F + k16, 16)]
                    plsc.addupdate(acc_s.at[pl.ds(le * F + k16, 16)],
                                   phi1 * ws1)
                    t0 = phi0 * ws0
                    for a in range(3):
                        w2u = web[e, pl.ds(2 * F + a * F + k16, 16)]
                        vja = ntb[e, pl.ds(3 * F + a * F + k16, 16)]
                        plsc.addupdate(
                            acc_v.at[pl.ds(le * (3 * F) + a * F + k16, 16)],
                            phi2 * w2u + t0 * vja)

        c = issue(0, nt_a, we_a, sem_a)
        c[0].wait()
        c[1].wait()

        def pair_body(jp, _):
            j = jp * 2
            cb = issue(j + 1, nt_b, we_b, sem_b)
            compute(j, nt_a, we_a)
            cb[0].wait()
            cb[1].wait()
            ca = issue(j + 2, nt_a, we_a, sem_a)
            compute(j + 1, nt_b, we_b)
            ca[0].wait()
            ca[1].wait()
            return 0

        lax.fori_loop(0, (nb + 1) // 2, pair_body, 0)

        pltpu.sync_copy(acc_s.at[pl.ds(0, NLOC * F)],
                        ds_hbm.at[pl.ds(segbase * F, NLOC * F)])
        pltpu.sync_copy(acc_v.at[pl.ds(0, NLOC * 3 * F)],
                        dv_hbm.at[pl.ds(segbase * 3 * F, NLOC * 3 * F)])
        return 0

    lax.fori_loop(0, PASSES, one_pass, 0)


def _sc_call(nt, we, dst, src):
    mesh = plsc.VectorSubcoreMesh(core_axis_name="c", subcore_axis_name="s")
    kfn = functools.partial(
        pl.kernel, mesh=mesh,
        out_type=[
            jax.ShapeDtypeStruct((NSEG * NLOC * F,), jnp.float32),
            jax.ShapeDtypeStruct((NSEG * NLOC * 3 * F,), jnp.float32),
        ],
        scratch_types=[
            pltpu.VMEM(((NLOC + 1) * F,), jnp.float32),
            pltpu.VMEM(((NLOC + 1) * 3 * F,), jnp.float32),
            pltpu.VMEM((CAP + 48,), jnp.int32),
            pltpu.VMEM((CAP + 48,), jnp.int32),
            pltpu.VMEM((CAP + 48,), jnp.int32),
            pltpu.VMEM((CH,), jnp.int32),
            pltpu.VMEM((CH,), jnp.int32),
            pltpu.VMEM((GB, 6 * F), jnp.float32),
            pltpu.VMEM((GB, 5 * F), jnp.float32),
            pltpu.VMEM((GB, 6 * F), jnp.float32),
            pltpu.VMEM((GB, 5 * F), jnp.float32),
            pltpu.SemaphoreType.DMA,
            pltpu.SemaphoreType.DMA,
        ],
    )(_sc_body)
    return kfn(nt, we, dst, src)


def kernel(s_j, v_j, r_ij, nbrs, W1, b1, W2, b2, Wd, bd):
    vjt = v_j.swapaxes(1, 2).reshape(N_NODES, 3 * F)
    nt = _node_table(s_j, vjt, W1, b1, W2, b2)
    we = _edge_table(r_ij, Wd, bd)
    dst = nbrs[:, 0]
    src = nbrs[:, 1]
    ds_flat, dv_flat = _sc_call(nt, we, dst, src)
    delta_s = ds_flat.reshape(NSEG * NLOC, F)[:N_NODES]
    delta_v = dv_flat.reshape(NSEG * NLOC, 3, F)[:N_NODES].swapaxes(1, 2)
    return (delta_s, delta_v)


# speedup vs baseline: 1.2882x; 1.1371x over previous
import functools

import jax
import jax.numpy as jnp
from jax import lax
from jax.experimental import pallas as pl
from jax.experimental.pallas import tpu as pltpu
from jax.experimental.pallas import tpu_sc as plsc

EPS = 1e-15
N_RBF = 20
CUTOFF = 5.0
F = 128

N_NODES = 10000
N_EDGES = 320000

NW = 32
PASSES = 3
NSEG = NW * PASSES
NLOC = 105
CAP = 4224
CH = 3200
GB = 16
WE = 4 * F


def _splat_gather(x, idx16):
    return x.at[idx16].get(mode="promise_in_bounds")


def _node_body(s_ref, vjt_ref, w1_ref, b1_ref, w2_ref, b2_ref, o_ref):
    s = s_ref[...]
    h = jnp.dot(s, w1_ref[...], preferred_element_type=jnp.float32) + b1_ref[...]
    h = h * jax.nn.sigmoid(h)
    phi = jnp.dot(h, w2_ref[...], preferred_element_type=jnp.float32) + b2_ref[...]
    o_ref[...] = jnp.concatenate([phi, vjt_ref[...]], axis=1)


def _node_table(s_j, vjt, W1, b1, W2, b2):
    n = s_j.shape[0]
    B = 2000
    return pl.pallas_call(
        _node_body,
        grid=(n // B,),
        in_specs=[
            pl.BlockSpec((B, F), lambda i: (i, 0)),
            pl.BlockSpec((B, 3 * F), lambda i: (i, 0)),
            pl.BlockSpec((F, F), lambda i: (0, 0)),
            pl.BlockSpec((1, F), lambda i: (0, 0)),
            pl.BlockSpec((F, 3 * F), lambda i: (0, 0)),
            pl.BlockSpec((1, 3 * F), lambda i: (0, 0)),
        ],
        out_specs=pl.BlockSpec((B, 6 * F), lambda i: (i, 0)),
        out_shape=jax.ShapeDtypeStruct((n, 6 * F), jnp.float32),
    )(s_j, vjt, W1, b1[None, :], W2, b2[None, :])


def _edge_body(r_ref, wd_ref, bd_ref, o_ref):
    r = r_ref[...]
    dist = jnp.sqrt(((r * r) + EPS).sum(-1))
    unit = r / dist[:, None]
    n = lax.iota(jnp.int32, N_RBF).astype(jnp.float32) + 1.0
    coef = n * (jnp.pi / CUTOFF)
    rbf = jnp.sin(coef * dist[:, None]) / dist[:, None]
    rbf_feats = jnp.dot(rbf, wd_ref[...], preferred_element_type=jnp.float32) + bd_ref[...]
    env = jnp.where(dist <= CUTOFF, 0.5 * (jnp.cos(jnp.pi / CUTOFF * dist) + 1.0), 0.0)
    w_s = rbf_feats * env[:, None]
    pad = jnp.zeros((r.shape[0], F - 3), jnp.float32)
    o_ref[...] = jnp.concatenate([w_s, unit, pad], axis=1)


def _edge_table(r_ij, Wd, bd):
    e = r_ij.shape[0]
    B = 2000
    return pl.pallas_call(
        _edge_body,
        grid=(e // B,),
        in_specs=[
            pl.BlockSpec((B, 3), lambda i: (i, 0)),
            pl.BlockSpec((N_RBF, 3 * F), lambda i: (0, 0)),
            pl.BlockSpec((1, 3 * F), lambda i: (0, 0)),
        ],
        out_specs=pl.BlockSpec((B, WE), lambda i: (i, 0)),
        out_shape=jax.ShapeDtypeStruct((e, WE), jnp.float32),
    )(r_ij, Wd, bd[None, :])


def _sc_body(nt_hbm, we_hbm, dst_hbm, src_hbm, ds_hbm, dv_hbm,
             acc_s, acc_v, pa_l, pb_l, st_dst, st_src,
             nt_a, we_a, nt_b, we_b, ia_w, ib_w, sem_a, sem_b):
    nc = 2
    wid = lax.axis_index("s") * nc + lax.axis_index("c")
    zs = jnp.zeros((16,), jnp.float32)
    zi = jnp.zeros((16,), jnp.int32)
    iota16 = lax.iota(jnp.int32, 16)
    LSZ = CAP + 48
    trash_pa = jnp.full((16,), NLOC << 19, jnp.int32)

    def pf_body(i, _):
        pa_l[pl.ds(i * 16, 16)] = trash_pa
        pb_l[pl.ds(i * 16, 16)] = zi
        return 0
    lax.fori_loop(0, 3 * LSZ // 16, pf_body, 0)

    lo = wid * (PASSES * NLOC)
    hi = lo + PASSES * NLOC

    def chunk_body(c, cnts):
        pltpu.sync_copy(dst_hbm.at[pl.ds(c * CH, CH)], st_dst)
        pltpu.sync_copy(src_hbm.at[pl.ds(c * CH, CH)], st_src)

        def grp_body(g, cnts):
            cnt0, cnt1, cnt2 = cnts
            for u in range(4):
                goff = (g * 4 + u) * 16
                dstv = st_dst[pl.ds(goff, 16)]
                lrel = dstv - lo
                m = (lrel >= 0) & (lrel < PASSES * NLOC)
                m0 = m & (lrel < NLOC)
                m1 = m & (lrel >= NLOC) & (lrel < 2 * NLOC)
                m2 = m & (lrel >= 2 * NLOC)
                w10 = jnp.where(m0, 1, 0).astype(jnp.int32)
                w11 = jnp.where(m1, 1, 0).astype(jnp.int32)
                w12 = jnp.where(m2, 1, 0).astype(jnp.int32)
                q0 = w10 + _splat_gather(w10, iota16 ^ 1)
                q0 = q0 + _splat_gather(q0, iota16 ^ 2)
                q1 = w11 + _splat_gather(w11, iota16 ^ 1)
                q1 = q1 + _splat_gather(q1, iota16 ^ 2)
                q2 = w12 + _splat_gather(w12, iota16 ^ 1)
                q2 = q2 + _splat_gather(q2, iota16 ^ 2)
                t0b = q0 + _splat_gather(q0, iota16 ^ 4)
                t0b = t0b + _splat_gather(t0b, iota16 ^ 8)
                t1b = q1 + _splat_gather(q1, iota16 ^ 4)
                t1b = t1b + _splat_gather(t1b, iota16 ^ 8)
                t2b = q2 + _splat_gather(q2, iota16 ^ 4)
                t2b = t2b + _splat_gather(t2b, iota16 ^ 8)
                qall = q0 + q1 + q2
                tall = t0b + t1b + t2b

                @pl.when(tall[0] > 0)
                def _(goff=goff, m=m, lrel=lrel, qall=qall,
                      w10=w10, w11=w11, w12=w12, q0=q0, q1=q1, q2=q2,
                      cnt0=cnt0, cnt1=cnt1, cnt2=cnt2):
                    srcv = st_src[pl.ds(goff, 16)]
                    eidv = iota16 + (c * CH + goff)
                    pav0 = eidv | (jnp.where(m0, lrel, NLOC) << 19)
                    pav1 = eidv | (jnp.where(m1, lrel - NLOC, NLOC) << 19)
                    pav2 = eidv | (jnp.where(m2, lrel - 2 * NLOC, NLOC) << 19)
                    off0 = cnt0[0]
                    off1 = cnt1[0] + LSZ
                    off2 = cnt2[0] + 2 * LSZ
                    for q in range(4):
                        @pl.when(qall[4 * q] > 0)
                        def _(q=q, off0=off0, off1=off1, off2=off2):
                            for j in range(4 * q, 4 * q + 4):
                                fj = jnp.full((16,), j, jnp.int32)

                                @pl.when(w10[j] == 1)
                                def _(off0=off0, fj=fj):
                                    pa_l[pl.ds(off0, 16)] = _splat_gather(pav0, fj)
                                    pb_l[pl.ds(off0, 16)] = _splat_gather(srcv, fj)

                                @pl.when(w11[j] == 1)
                                def _(off1=off1, fj=fj):
                                    pa_l[pl.ds(off1, 16)] = _splat_gather(pav1, fj)
                                    pb_l[pl.ds(off1, 16)] = _splat_gather(srcv, fj)

                                @pl.when(w12[j] == 1)
                                def _(off2=off2, fj=fj):
                                    pa_l[pl.ds(off2, 16)] = _splat_gather(pav2, fj)
                                    pb_l[pl.ds(off2, 16)] = _splat_gather(srcv, fj)
                                off0 = off0 + w10[j]
                                off1 = off1 + w11[j]
                                off2 = off2 + w12[j]
                        off0 = off0 + q0[4 * q]
                        off1 = off1 + q1[4 * q]
                        off2 = off2 + q2[4 * q]
                    pa_l[pl.ds(off0, 16)] = trash_pa
                    pb_l[pl.ds(off0, 16)] = zi
                    pa_l[pl.ds(off1, 16)] = trash_pa
                    pb_l[pl.ds(off1, 16)] = zi
                    pa_l[pl.ds(off2, 16)] = trash_pa
                    pb_l[pl.ds(off2, 16)] = zi
                cnt0 = jnp.minimum(cnt0 + t0b, CAP - 16)
                cnt1 = jnp.minimum(cnt1 + t1b, CAP - 16)
                cnt2 = jnp.minimum(cnt2 + t2b, CAP - 16)
            return (cnt0, cnt1, cnt2)

        return lax.fori_loop(0, CH // 64, grp_body, cnts)

    cnts = lax.fori_loop(0, N_EDGES // CH, chunk_body, (zi, zi, zi))

    def one_pass(p, _):
        segbase = (wid * PASSES + p) * NLOC
        pof = p * LSZ
        cs = jnp.where(p == 0, cnts[0][0],
                       jnp.where(p == 1, cnts[1][0], cnts[2][0]))
        nb = (cs + (GB - 1)) // GB

        def zs_body(i, _):
            acc_s[pl.ds(i * 16, 16)] = zs
            return 0
        lax.fori_loop(0, (NLOC + 1) * F // 16, zs_body, 0)

        def zv_body(i, _):
            acc_v[pl.ds(i * 16, 16)] = zs
            return 0
        lax.fori_loop(0, (NLOC + 1) * 3 * F // 16, zv_body, 0)

        def issue(j, ntb, web, iw, sem):
            j0 = pof + j * GB
            pav = pa_l[pl.ds(j0, GB)]
            iw[pl.ds(0, 16)] = pav & ((1 << 19) - 1)
            c1 = pltpu.async_copy(nt_hbm.at[pb_l.at[pl.ds(j0, GB)]], ntb, sem)
            c2 = pltpu.async_copy(we_hbm.at[iw.at[pl.ds(0, GB)]], web, sem)
            return c1, c2

        def compute(j, ntb, web):
            pav = pa_l[pl.ds(pof + j * GB, GB)]
            locv = pav >> 19
            for e in range(GB):
                le = locv[e]
                uvec = web[e, pl.ds(3 * F, 16)]
                u0, u1, u2 = uvec[0], uvec[1], uvec[2]
                for k in range(8):
                    k16 = k * 16
                    phi0 = ntb[e, pl.ds(k16, 16)]
                    phi1 = ntb[e, pl.ds(F + k16, 16)]
                    phi2 = ntb[e, pl.ds(2 * F + k16, 16)]
                    ws0 = web[e, pl.ds(k16, 16)]
                    ws1 = web[e, pl.ds(F + k16, 16)]
                    ws2 = web[e, pl.ds(2 * F + k16, 16)]
                    plsc.addupdate(acc_s.at[pl.ds(le * F + k16, 16)],
                                   phi1 * ws1)
                    t0 = phi0 * ws0
                    t2 = phi2 * ws2
                    for a, ua in ((0, u0), (1, u1), (2, u2)):
                        vja = ntb[e, pl.ds(3 * F + a * F + k16, 16)]
                        plsc.addupdate(
                            acc_v.at[pl.ds(le * (3 * F) + a * F + k16, 16)],
                            t2 * ua + t0 * vja)

        c = issue(0, nt_a, we_a, ia_w, sem_a)
        c[0].wait()
        c[1].wait()

        def pair_body(jp, _):
            j = jp * 2
            cb = issue(j + 1, nt_b, we_b, ib_w, sem_b)
            compute(j, nt_a, we_a)
            cb[0].wait()
            cb[1].wait()
            ca = issue(j + 2, nt_a, we_a, ia_w, sem_a)
            compute(j + 1, nt_b, we_b)
            ca[0].wait()
            ca[1].wait()
            return 0

        lax.fori_loop(0, (nb + 1) // 2, pair_body, 0)

        pltpu.sync_copy(acc_s.at[pl.ds(0, NLOC * F)],
                        ds_hbm.at[pl.ds(segbase * F, NLOC * F)])
        pltpu.sync_copy(acc_v.at[pl.ds(0, NLOC * 3 * F)],
                        dv_hbm.at[pl.ds(segbase * 3 * F, NLOC * 3 * F)])
        return 0

    lax.fori_loop(0, PASSES, one_pass, 0)


def _sc_call(nt, we, dst, src):
    mesh = plsc.VectorSubcoreMesh(core_axis_name="c", subcore_axis_name="s")
    kfn = functools.partial(
        pl.kernel, mesh=mesh,
        out_type=[
            jax.ShapeDtypeStruct((NSEG * NLOC * F,), jnp.float32),
            jax.ShapeDtypeStruct((NSEG * NLOC * 3 * F,), jnp.float32),
        ],
        scratch_types=[
            pltpu.VMEM(((NLOC + 1) * F,), jnp.float32),
            pltpu.VMEM(((NLOC + 1) * 3 * F,), jnp.float32),
            pltpu.VMEM((3 * (CAP + 48),), jnp.int32),
            pltpu.VMEM((3 * (CAP + 48),), jnp.int32),
            pltpu.VMEM((CH,), jnp.int32),
            pltpu.VMEM((CH,), jnp.int32),
            pltpu.VMEM((GB, 6 * F), jnp.float32),
            pltpu.VMEM((GB, WE), jnp.float32),
            pltpu.VMEM((GB, 6 * F), jnp.float32),
            pltpu.VMEM((GB, WE), jnp.float32),
            pltpu.VMEM((16,), jnp.int32),
            pltpu.VMEM((16,), jnp.int32),
            pltpu.SemaphoreType.DMA,
            pltpu.SemaphoreType.DMA,
        ],
    )(_sc_body)
    return kfn(nt, we, dst, src)


def kernel(s_j, v_j, r_ij, nbrs, W1, b1, W2, b2, Wd, bd):
    vjt = v_j.swapaxes(1, 2).reshape(N_NODES, 3 * F)
    nt = _node_table(s_j, vjt, W1, b1, W2, b2)
    we = _edge_table(r_ij, Wd, bd)
    dst = nbrs[:, 0]
    src = nbrs[:, 1]
    ds_flat, dv_flat = _sc_call(nt, we, dst, src)
    delta_s = ds_flat.reshape(NSEG * NLOC, F)[:N_NODES]
    delta_v = dv_flat.reshape(NSEG * NLOC, 3, F)[:N_NODES].swapaxes(1, 2)
    return (delta_s, delta_v)
